# Initial kernel scaffold; baseline (speedup 1.0000x reference)
import jax
import jax.numpy as jnp
from jax.experimental import pallas as pl


def kernel(x, edge_index, edge_label, W1, b1, W2, b2):
    raise NotImplementedError("write your pallas kernel here")


# speedup vs baseline: 3.6100x; 3.6100x over previous
import functools

import jax
import jax.numpy as jnp
from jax import lax
from jax.experimental import pallas as pl
from jax.experimental.pallas import tpu as pltpu
from jax.experimental.pallas import tpu_sc as plsc

N_NODES = 10000
D = 128
N_EDGES = 320000

NW = 32
CHUNK = 128
NCHUNK = 79
PER_W = CHUNK * NCHUNK
E_PAD = PER_W * NW

ROW_BLOCK = 1000


def _mlp_body(x_ref, w1_ref, b1_ref, w2_ref, b2_ref, h_ref):
    h1 = jnp.dot(x_ref[...], w1_ref[...], preferred_element_type=jnp.float32)
    h1 = jnp.maximum(h1 + b1_ref[...], 0.0)
    h2 = jnp.dot(h1, w2_ref[...], preferred_element_type=jnp.float32)
    h_ref[...] = jnp.maximum(h2 + b2_ref[...], 0.0)


def _mlp(x, W1, b1, W2, b2):
    grid = (N_NODES // ROW_BLOCK,)
    return pl.pallas_call(
        _mlp_body,
        grid=grid,
        in_specs=[
            pl.BlockSpec((ROW_BLOCK, D), lambda i: (i, 0)),
            pl.BlockSpec((D, D), lambda i: (0, 0)),
            pl.BlockSpec((1, D), lambda i: (0, 0)),
            pl.BlockSpec((D, D), lambda i: (0, 0)),
            pl.BlockSpec((1, D), lambda i: (0, 0)),
        ],
        out_specs=pl.BlockSpec((ROW_BLOCK, D), lambda i: (i, 0)),
        out_shape=jax.ShapeDtypeStruct((N_NODES, D), jnp.float32),
    )(x, W1, b1.reshape(1, D), W2, b2.reshape(1, D))


def _edge_dot_body(h_hbm, src_hbm, dst_hbm, pred_hbm,
                   sidx, didx, srows, drows, accbuf, outbuf, sem1, sem2):
    wid = lax.axis_index("s") * 2 + lax.axis_index("c")
    base = wid * PER_W
    lanes = lax.iota(jnp.int32, 16)

    def chunk_body(c, carry):
        off = base + c * CHUNK
        pltpu.sync_copy(src_hbm.at[pl.ds(off, CHUNK)], sidx)
        pltpu.sync_copy(dst_hbm.at[pl.ds(off, CHUNK)], didx)
        cp1 = pltpu.async_copy(h_hbm.at[sidx], srows, sem1)
        cp2 = pltpu.async_copy(h_hbm.at[didx], drows, sem2)
        cp1.wait()
        cp2.wait()

        def group_body(g, gcarry):
            ebase = g * 16
            for r in range(16):
                e = ebase + r
                acc = srows[e, pl.ds(0, 16)] * drows[e, pl.ds(0, 16)]
                for j in range(1, D // 16):
                    acc = acc + srows[e, pl.ds(j * 16, 16)] * drows[e, pl.ds(j * 16, 16)]
                accbuf[pl.ds(r * 16, 16)] = acc
            rowbase = lanes * 16
            out16 = plsc.load_gather(accbuf, [rowbase])
            for l in range(1, 16):
                out16 = out16 + plsc.load_gather(accbuf, [rowbase + l])
            outbuf[pl.ds(ebase, 16)] = out16
            return gcarry

        lax.fori_loop(0, CHUNK // 16, group_body, 0)
        pltpu.sync_copy(outbuf, pred_hbm.at[pl.ds(off, CHUNK)])
        return carry

    lax.fori_loop(0, NCHUNK, chunk_body, 0)


def _edge_dots(h, src, dst):
    mesh = plsc.VectorSubcoreMesh(core_axis_name="c", subcore_axis_name="s")
    k = functools.partial(
        pl.kernel,
        out_type=jax.ShapeDtypeStruct((E_PAD,), jnp.float32),
        mesh=mesh,
        scratch_types=[
            pltpu.VMEM((CHUNK,), jnp.int32),
            pltpu.VMEM((CHUNK,), jnp.int32),
            pltpu.VMEM((CHUNK, D), jnp.float32),
            pltpu.VMEM((CHUNK, D), jnp.float32),
            pltpu.VMEM((256,), jnp.float32),
            pltpu.VMEM((CHUNK,), jnp.float32),
            pltpu.SemaphoreType.DMA,
            pltpu.SemaphoreType.DMA,
        ],
        compiler_params=pltpu.CompilerParams(needs_layout_passes=False),
    )(_edge_dot_body)
    return k(h, src, dst)


def kernel(x, edge_index, edge_label, W1, b1, W2, b2):
    h = _mlp(x, W1, b1, W2, b2)
    ei = edge_index.astype(jnp.int32)
    pad = E_PAD - N_EDGES
    src = jnp.pad(ei[0], (0, pad))
    dst = jnp.pad(ei[1], (0, pad))
    pred_pad = _edge_dots(h, src, dst)
    return (pred_pad[:N_EDGES], edge_label)
